# baseline (speedup 1.0000x reference)
import functools

import jax
import jax.numpy as jnp
from jax import lax
from jax.experimental import pallas as pl
from jax.experimental.pallas import tpu as pltpu
from jax.experimental.pallas import tpu_sc as plsc

VOCAB = 1000000
HIDDEN = 64
MAX_POS = 20
BATCH = 16384
EPS = 1e-05

NC = 2
NS = 16
NW = NC * NS

ROWS = BATCH * MAX_POS
ROWS_PER_W = ROWS // NW
JBLK = 128
CHUNK = 640
NJ = CHUNK // JBLK
NCHUNK = ROWS_PER_W // CHUNK
GROUPS = CHUNK // MAX_POS
NV = HIDDEN // 16


def _rsqrt_newton(v):
    i = lax.bitcast_convert_type(v, jnp.int32)
    i = jnp.int32(0x5F3759DF) - lax.shift_right_logical(i, 1)
    y = lax.bitcast_convert_type(i, jnp.float32)
    half = jnp.float32(0.5) * v
    for _ in range(3):
        y = y * (jnp.float32(1.5) - half * y * y)
    return y


def _lane_sum(v, iota):
    for d in (8, 4, 2, 1):
        idx = lax.bitwise_xor(iota, jnp.int32(d))
        v = v + v.at[idx].get(mode="promise_in_bounds")
    return v


def _body(ids_hbm, table_hbm, pos_hbm, gamma_hbm, beta_hbm, out_hbm,
          idx_v, rows_v, pos_v, gam_v, bet_v, sem, osem):
    wid = lax.axis_index("s") * NC + lax.axis_index("c")
    base = wid * ROWS_PER_W

    pltpu.sync_copy(pos_hbm, pos_v)
    pltpu.sync_copy(gamma_hbm, gam_v)
    pltpu.sync_copy(beta_hbm, bet_v)

    gvec = [gam_v[pl.ds(16 * k, 16)] for k in range(NV)]
    bvec = [bet_v[pl.ds(16 * k, 16)] for k in range(NV)]

    inv_h = jnp.float32(1.0 / HIDDEN)
    inv_hm1 = jnp.float32(1.0 / (HIDDEN - 1))
    eps = jnp.float32(EPS)
    iota = lax.iota(jnp.int32, 16)

    @pl.loop(0, NCHUNK)
    def _chunk(c):
        row0 = base + c * CHUNK
        pltpu.sync_copy(ids_hbm.at[pl.ds(row0, CHUNK)], idx_v)
        copies = [
            pltpu.async_copy(table_hbm.at[idx_v.at[pl.ds(j * JBLK, JBLK)]],
                             rows_v.at[pl.ds(j * JBLK, JBLK)], sem)
            for j in range(NJ)
        ]
        for cp in copies:
            cp.wait()

        @pl.loop(0, GROUPS)
        def _group(g):
            r0 = g * MAX_POS
            for l in range(MAX_POS):
                r = r0 + l
                x = [rows_v[r, pl.ds(16 * k, 16)] + pos_v[l, pl.ds(16 * k, 16)]
                     for k in range(NV)]
                s = (x[0] + x[1]) + (x[2] + x[3])
                sq = (x[0] * x[0] + x[1] * x[1]) + (x[2] * x[2] + x[3] * x[3])
                tot = _lane_sum(s, iota)
                tot2 = _lane_sum(sq, iota)
                mean = tot * inv_h
                var = jnp.maximum((tot2 - tot * mean) * inv_hm1,
                                  jnp.float32(0.0))
                std = var * _rsqrt_newton(var)
                inv = jnp.float32(1.0) / (std + eps)
                for k in range(NV):
                    rows_v[r, pl.ds(16 * k, 16)] = (x[k] - mean) * inv * gvec[k] + bvec[k]

        pltpu.async_copy(rows_v, out_hbm.at[pl.ds(row0, CHUNK)], osem).wait()


@functools.partial(
    pl.kernel,
    out_type=jax.ShapeDtypeStruct((ROWS, HIDDEN), jnp.float32),
    mesh=plsc.VectorSubcoreMesh(core_axis_name="c", subcore_axis_name="s"),
    scratch_types=[
        pltpu.VMEM((CHUNK,), jnp.int32),
        pltpu.VMEM((CHUNK, HIDDEN), jnp.float32),
        pltpu.VMEM((MAX_POS, HIDDEN), jnp.float32),
        pltpu.VMEM((HIDDEN,), jnp.float32),
        pltpu.VMEM((HIDDEN,), jnp.float32),
        pltpu.SemaphoreType.DMA,
        pltpu.SemaphoreType.DMA,
    ],
    compiler_params=pltpu.CompilerParams(use_tc_tiling_on_sc=False),
)
def _embed_ln(*args):
    _body(*args)


def kernel(input_ids, table, pos_table, gamma, beta):
    ids1d = input_ids.astype(jnp.int32).reshape(ROWS)
    out = _embed_ln(ids1d, table, pos_table, gamma, beta)
    return out.reshape(BATCH, MAX_POS, HIDDEN)
